# Initial kernel scaffold; baseline (speedup 1.0000x reference)
import jax
import jax.numpy as jnp
from jax.experimental import pallas as pl


def kernel(x_content, y_content, means_table, stds_table):
    raise NotImplementedError("write your pallas kernel here")


# speedup vs baseline: 18.8822x; 18.8822x over previous
import functools

import jax
import jax.numpy as jnp
from jax.experimental import pallas as pl
from jax.experimental.pallas import tpu as pltpu

NUM_CLASSES = 19
KP = 24
EPS = 1e-05
COUNT = 6
B, C, H, W = 4, 96, 224, 224
HW = H * W
T = 7168
NB = HW // T


def _stats_body(x_ref, lab_ref, mt_ref, st_ref, sb_ref, tab_ref, used_ref,
                acc_ref):
    j = pl.program_id(1)
    x = x_ref[0]
    lab = lab_ref[0]
    klass = jax.lax.broadcasted_iota(jnp.int32, (KP, T), 0)
    onehot = (klass == lab).astype(jnp.float32)
    x3 = jnp.concatenate([x, x * x, jnp.ones((8, T), jnp.float32)], axis=0)
    part = jax.lax.dot_general(
        x3, onehot, (((1,), (1,)), ((), ())),
        preferred_element_type=jnp.float32,
        precision=jax.lax.Precision.HIGHEST)

    @pl.when(j == 0)
    def _():
        acc_ref[...] = part

    @pl.when(j > 0)
    def _():
        acc_ref[...] += part

    @pl.when(j == NB - 1)
    def _():
        stats = acc_ref[...]
        sums = stats[:C, :]
        sumsq = stats[C:2 * C, :]
        cnt = stats[2 * C:2 * C + 1, :]
        mean = sums / jnp.maximum(cnt, 1.0)
        var = (sumsq - cnt * mean * mean) / jnp.maximum(cnt - 1.0, 1.0)
        std = jnp.sqrt(jnp.maximum(var, 0.0)) + EPS
        used = cnt > float(COUNT)
        mt = mt_ref[...]
        st = st_ref[...]
        scale = jnp.where(used, st / std, 1.0)
        bias = jnp.where(used, mt - mean * scale, 0.0)
        sb_ref[0, 0] = scale
        sb_ref[0, 1] = bias
        tab_ref[0, 0] = jnp.where(used, mt, 0.0)
        tab_ref[0, 1] = jnp.where(used, st, 0.0)
        used_ref[0] = used.astype(jnp.int32)


def _apply_body(x_ref, lab_ref, sb_ref, out_ref):
    x = x_ref[0]
    lab = lab_ref[0]
    klass = jax.lax.broadcasted_iota(jnp.int32, (KP, T), 0)
    onehot = (klass == lab).astype(jnp.float32)
    scale = sb_ref[0, 0]
    bias = sb_ref[0, 1]
    scale_px = jax.lax.dot_general(
        scale, onehot, (((1,), (0,)), ((), ())),
        preferred_element_type=jnp.float32,
        precision=jax.lax.Precision.HIGHEST)
    bias_px = jax.lax.dot_general(
        bias, onehot, (((1,), (0,)), ((), ())),
        preferred_element_type=jnp.float32,
        precision=jax.lax.Precision.HIGHEST)
    out_ref[0] = x * scale_px + bias_px


@jax.jit
def kernel(x_content, y_content, means_table, stds_table):
    xf = x_content.reshape(B, C, HW)
    labs = y_content.reshape(B * NB, 1, T)
    mt_t = jnp.zeros((C, KP), jnp.float32).at[:, :NUM_CLASSES].set(means_table.T)
    st_t = jnp.zeros((C, KP), jnp.float32).at[:, :NUM_CLASSES].set(stds_table.T)

    sb, tab, used_i = pl.pallas_call(
        _stats_body,
        grid=(B, NB),
        in_specs=[
            pl.BlockSpec((1, C, T), lambda b, j: (b, 0, j)),
            pl.BlockSpec((1, 1, T), lambda b, j: (b * NB + j, 0, 0)),
            pl.BlockSpec((C, KP), lambda b, j: (0, 0)),
            pl.BlockSpec((C, KP), lambda b, j: (0, 0)),
        ],
        out_specs=[
            pl.BlockSpec((1, 2, C, KP), lambda b, j: (b, 0, 0, 0)),
            pl.BlockSpec((1, 2, C, KP), lambda b, j: (b, 0, 0, 0)),
            pl.BlockSpec((1, 1, KP), lambda b, j: (b, 0, 0)),
        ],
        out_shape=[
            jax.ShapeDtypeStruct((B, 2, C, KP), jnp.float32),
            jax.ShapeDtypeStruct((B, 2, C, KP), jnp.float32),
            jax.ShapeDtypeStruct((B, 1, KP), jnp.int32),
        ],
        scratch_shapes=[pltpu.VMEM((2 * C + 8, KP), jnp.float32)],
        compiler_params=pltpu.CompilerParams(
            dimension_semantics=("arbitrary", "arbitrary")),
    )(xf, labs, mt_t, st_t)

    out_flat = pl.pallas_call(
        _apply_body,
        grid=(B, NB),
        in_specs=[
            pl.BlockSpec((1, C, T), lambda b, j: (b, 0, j)),
            pl.BlockSpec((1, 1, T), lambda b, j: (b * NB + j, 0, 0)),
            pl.BlockSpec((1, 2, C, KP), lambda b, j: (b, 0, 0, 0)),
        ],
        out_specs=pl.BlockSpec((1, C, T), lambda b, j: (b, 0, j)),
        out_shape=jax.ShapeDtypeStruct((B, C, HW), jnp.float32),
        compiler_params=pltpu.CompilerParams(
            dimension_semantics=("parallel", "arbitrary")),
    )(xf, labs, sb)

    out = out_flat.reshape(B, C, H, W)
    sm = tab[:, 0].transpose(0, 2, 1)[:, :NUM_CLASSES, :]
    ss = tab[:, 1].transpose(0, 2, 1)[:, :NUM_CLASSES, :]
    used = used_i[:, 0, :NUM_CLASSES] != 0
    return out, sm, ss, used
